# baseline (speedup 1.0000x reference)
import functools

import jax
import jax.numpy as jnp
from jax import lax
from jax.experimental import pallas as pl
from jax.experimental.pallas import tpu as pltpu
from jax.experimental.pallas import tpu_sc as plsc

NC = 2
NS = 16
NW = NC * NS
CHUNK = 1000


def _round_up(n, m):
    return (n + m - 1) // m * m


@functools.lru_cache(maxsize=None)
def _build(b_node_pad, b_edge_pad, dim):
    n_w_n = b_node_pad // NW
    n_w_e = b_edge_pad // NW
    idx_slab = max(n_w_n, n_w_e)
    mesh = plsc.VectorSubcoreMesh(
        core_axis_name="c", subcore_axis_name="s", num_cores=NC, num_subcores=NS
    )

    @functools.partial(
        pl.kernel,
        mesh=mesh,
        compiler_params=pltpu.CompilerParams(
            use_tc_tiling_on_sc=False, skip_device_barrier=True
        ),
        out_type=[
            jax.ShapeDtypeStruct((b_node_pad, dim), jnp.float32),
            jax.ShapeDtypeStruct((b_edge_pad, dim), jnp.float32),
        ],
        scratch_types=[
            pltpu.VMEM((idx_slab,), jnp.int32),
            pltpu.VMEM((2, CHUNK, dim), jnp.float32),
            pltpu.SemaphoreType.DMA,
            pltpu.SemaphoreType.DMA,
            pltpu.SemaphoreType.DMA,
            pltpu.SemaphoreType.DMA,
        ],
    )
    def emb_kernel(x_hbm, e_hbm, ntab, etab, out_n, out_e,
                   idx_s, rows_v, g0, g1, w0, w1):
        wid = lax.axis_index("s") * NC + lax.axis_index("c")
        sem_g = (g0, g1)
        sem_w = (w0, w1)

        def phase(idx_hbm, tab_hbm, out_hbm, n_w):
            n = n_w // CHUNK
            base = wid * n_w
            pltpu.sync_copy(idx_hbm.at[pl.ds(base, n_w)], idx_s.at[pl.ds(0, n_w)])

            def g_start(i, b):
                off = pl.multiple_of(i * CHUNK, 8)
                pltpu.async_copy(
                    tab_hbm.at[idx_s.at[pl.ds(off, CHUNK)]], rows_v.at[b], sem_g[b]
                )

            def g_wait(b):
                pltpu.make_async_copy(
                    tab_hbm.at[idx_s.at[pl.ds(0, CHUNK)]], rows_v.at[b], sem_g[b]
                ).wait()

            def w_start(i, b):
                off = pl.multiple_of(base + i * CHUNK, 8)
                pltpu.async_copy(rows_v.at[b], out_hbm.at[pl.ds(off, CHUNK)], sem_w[b])

            def w_wait(b):
                pltpu.make_async_copy(
                    rows_v.at[b], out_hbm.at[pl.ds(base, CHUNK)], sem_w[b]
                ).wait()

            g_start(0, 0)
            g_start(1, 1)

            @pl.loop(0, n, step=2)
            def _(i):
                for b in range(2):
                    ib = i + b
                    g_wait(b)
                    w_start(ib, b)
                    w_wait(b)

                    @pl.when(ib + 2 < n)
                    def _():
                        g_start(ib + 2, b)

        phase(e_hbm, etab, out_e, n_w_e)
        phase(x_hbm, ntab, out_n, n_w_n)

    return emb_kernel


def kernel(x, edge_attr, node_table, edge_table):
    b_n = x.shape[0]
    b_e = edge_attr.shape[0]
    dim = node_table.shape[1]
    b_n_pad = _round_up(b_n, NW * CHUNK * 2)
    b_e_pad = _round_up(b_e, NW * CHUNK * 2)
    x_i = jnp.pad(x.astype(jnp.int32), (0, b_n_pad - b_n))
    e_i = jnp.pad(edge_attr.astype(jnp.int32), (0, b_e_pad - b_e))
    out_n, out_e = _build(b_n_pad, b_e_pad, dim)(x_i, e_i, node_table, edge_table)
    return (out_n[:b_n], out_e[:b_e])


# speedup vs baseline: 1.0131x; 1.0131x over previous
import functools

import jax
import jax.numpy as jnp
from jax import lax
from jax.experimental import pallas as pl
from jax.experimental.pallas import tpu as pltpu
from jax.experimental.pallas import tpu_sc as plsc

NC = 2
NS = 16
NW = NC * NS
CHUNK = 1000


def _round_up(n, m):
    return (n + m - 1) // m * m


@functools.lru_cache(maxsize=None)
def _build(b_node_pad, b_edge_pad, vocab, dim):
    n_w_n = b_node_pad // NW
    n_w_e = b_edge_pad // NW
    rows128 = CHUNK * dim // 128
    mesh = plsc.VectorSubcoreMesh(
        core_axis_name="c", subcore_axis_name="s", num_cores=NC, num_subcores=NS
    )

    @functools.partial(
        pl.kernel,
        mesh=mesh,
        compiler_params=pltpu.CompilerParams(use_tc_tiling_on_sc=False),
        out_type=[
            jax.ShapeDtypeStruct((b_node_pad * dim // 128, 128), jnp.float32),
            jax.ShapeDtypeStruct((b_edge_pad * dim // 128, 128), jnp.float32),
        ],
        scratch_types=[
            pltpu.VMEM((2, CHUNK), jnp.int32),
            pltpu.VMEM((2, CHUNK, dim), jnp.float32),
            pltpu.VMEM((rows128, 128), jnp.float32),
            pltpu.SemaphoreType.DMA,
            pltpu.SemaphoreType.DMA,
            pltpu.SemaphoreType.DMA,
            pltpu.SemaphoreType.DMA,
            pltpu.SemaphoreType.DMA,
        ],
    )
    def emb_kernel(x_hbm, e_hbm, ntab, etab, out_n1, out_e1,
                   idx_v, rows_v, stage_v, i0, i1, g0, g1, wsem):
        wid = lax.axis_index("s") * NC + lax.axis_index("c")
        sem_i = (i0, i1)
        sem_g = (g0, g1)

        def phase(idx_hbm, tab_hbm, out_hbm, n_w, first):
            n = n_w // CHUNK
            base = wid * n_w
            obase = base * dim // 128

            def ix_start(i, b):
                off = pl.multiple_of(base + i * CHUNK, 8)
                pltpu.async_copy(
                    idx_hbm.at[pl.ds(off, CHUNK)], idx_v.at[b], sem_i[b]
                )

            def ix_wait(b):
                pltpu.make_async_copy(
                    idx_hbm.at[pl.ds(base, CHUNK)], idx_v.at[b], sem_i[b]
                ).wait()

            def g_start(b):
                pltpu.async_copy(tab_hbm.at[idx_v.at[b]], rows_v.at[b], sem_g[b])

            def g_wait(b):
                pltpu.make_async_copy(
                    tab_hbm.at[idx_v.at[b]], rows_v.at[b], sem_g[b]
                ).wait()

            def w_start(i):
                pltpu.async_copy(
                    stage_v, out_hbm.at[pl.ds(obase + i * rows128, rows128)], wsem
                )

            def w_wait():
                pltpu.make_async_copy(
                    stage_v, out_hbm.at[pl.ds(obase, rows128)], wsem
                ).wait()

            def repack(b):
                @pl.loop(0, rows128)
                def _(jj):
                    r4 = jj * 4
                    for q in range(4):
                        for h in range(2):
                            v = rows_v[b, r4 + q, pl.ds(h * 16, 16)]
                            stage_v[jj, pl.ds(q * 32 + h * 16, 16)] = v

            ix_start(0, 0)
            ix_start(1, 1)
            ix_wait(0)
            g_start(0)

            @pl.loop(0, n, step=2)
            def _(i):
                for b in range(2):
                    ib = i + b
                    bo = 1 - b

                    @pl.when(ib + 1 < n)
                    def _():
                        ix_wait(bo)
                        g_start(bo)
                    g_wait(b)

                    @pl.when(ib + 2 < n)
                    def _():
                        ix_start(ib + 2, b)
                    if first:
                        @pl.when(ib >= 1)
                        def _():
                            w_wait()
                    else:
                        w_wait()
                    repack(b)
                    w_start(ib)

        phase(e_hbm, etab, out_e1, n_w_e, True)
        phase(x_hbm, ntab, out_n1, n_w_n, False)
        pltpu.make_async_copy(
            stage_v, out_n1.at[pl.ds(0, rows128)], wsem
        ).wait()

    return emb_kernel


def kernel(x, edge_attr, node_table, edge_table):
    b_n = x.shape[0]
    b_e = edge_attr.shape[0]
    vocab, dim = node_table.shape
    b_n_pad = _round_up(b_n, NW * CHUNK * 2)
    b_e_pad = _round_up(b_e, NW * CHUNK * 2)
    x_i = jnp.pad(x.astype(jnp.int32), (0, b_n_pad - b_n))
    e_i = jnp.pad(edge_attr.astype(jnp.int32), (0, b_e_pad - b_e))
    out_n1, out_e1 = _build(b_n_pad, b_e_pad, vocab, dim)(
        x_i, e_i, node_table, edge_table
    )
    out_n = out_n1.reshape(b_n_pad, dim)[:b_n]
    out_e = out_e1.reshape(b_e_pad, dim)[:b_e]
    return (out_n, out_e)
